# baseline (speedup 1.0000x reference)
import jax
import jax.numpy as jnp
from jax.experimental import pallas as pl

_BB = 256


def _body(x_ref, masked_ref, idx_ref):
    x = x_ref[...]
    norms = jnp.sqrt(jnp.sum(x * x, axis=2))
    idx = jnp.argmax(norms, axis=1).astype(jnp.int32)
    onehot = (
        jax.lax.broadcasted_iota(jnp.int32, norms.shape, 1) == idx[:, None]
    ).astype(x.dtype)
    sel = jnp.sum(x * onehot[:, :, None], axis=1)
    masked_ref[...] = sel[:, None, :]
    idx_ref[...] = idx


def kernel(x):
    B, C, D = x.shape
    grid = (B // _BB,)
    masked, idx = pl.pallas_call(
        _body,
        grid=grid,
        in_specs=[pl.BlockSpec((_BB, C, D), lambda i: (i, 0, 0))],
        out_specs=[
            pl.BlockSpec((_BB, 1, D), lambda i: (i, 0, 0)),
            pl.BlockSpec((_BB,), lambda i: (i,)),
        ],
        out_shape=[
            jax.ShapeDtypeStruct((B, 1, D), x.dtype),
            jax.ShapeDtypeStruct((B,), jnp.int32),
        ],
    )(x)
    return masked, idx
